# baseline (speedup 1.0000x reference)
import functools

import jax
import jax.numpy as jnp
from jax import lax
from jax.experimental import pallas as pl
from jax.experimental.pallas import tpu as pltpu
from jax.experimental.pallas import tpu_sc as plsc

N = 10000
E = 320000
F_IN = 128
H = 8
C = 8
HC = 64
D2 = 48
NCLASS = 40
RB = 1000
NRB = N // RB
NTILE = 32
EPT = 9984
CK = 128
NCHUNK = EPT // CK
RSTAGE = 1000

f32 = jnp.float32
i32 = jnp.int32


def _leaky(x):
    return jnp.where(x > 0, x, 0.2 * x)


def _elu(x):
    return jnp.where(x > 0, x, 0.2 * (jnp.exp(x) - 1.0))


def _prep1_body(x_ref, w_ref, bt1_ref, r8_ref,
                h_ref, t1_ref, ninit_ref, dinit_ref):
    h = jnp.dot(x_ref[0], w_ref[0], preferred_element_type=f32)
    t1 = jnp.dot(h, bt1_ref[0], preferred_element_type=f32)
    exl = jnp.exp(_leaky(t1[:, :8] + t1[:, 8:]))
    h_ref[0] = h
    t1_ref[0] = t1
    ninit_ref[0] = h * jnp.dot(exl, r8_ref[...], preferred_element_type=f32)
    dinit_ref[0] = jnp.concatenate([exl, exl], axis=1)


def _prep1(x_list, W1, BT1, R8):
    return pl.pallas_call(
        _prep1_body,
        grid=(2, NRB),
        in_specs=[
            pl.BlockSpec((1, RB, F_IN), lambda k, i: (k, i, 0)),
            pl.BlockSpec((1, F_IN, HC), lambda k, i: (k, 0, 0)),
            pl.BlockSpec((1, HC, 16), lambda k, i: (k, 0, 0)),
            pl.BlockSpec((H, HC), lambda k, i: (0, 0)),
        ],
        out_specs=[
            pl.BlockSpec((1, RB, HC), lambda k, i: (k, i, 0)),
            pl.BlockSpec((1, RB, 16), lambda k, i: (k, i, 0)),
            pl.BlockSpec((1, RB, HC), lambda k, i: (k, i, 0)),
            pl.BlockSpec((1, RB, 16), lambda k, i: (k, i, 0)),
        ],
        out_shape=[
            jax.ShapeDtypeStruct((2, N, HC), f32),
            jax.ShapeDtypeStruct((2, N, 16), f32),
            jax.ShapeDtypeStruct((2, N, HC), f32),
            jax.ShapeDtypeStruct((2, N, 16), f32),
        ],
    )(x_list, W1, BT1, R8)


def _make_edge_kernel(D, headed, dual_t):
    NV = D // 16
    mesh = plsc.VectorSubcoreMesh(core_axis_name="c", subcore_axis_name="s")

    scratch = [
        pltpu.VMEM_SHARED((N, D), f32),
        pltpu.VMEM_SHARED((N, 16), f32),
    ]
    if dual_t:
        scratch.append(pltpu.VMEM_SHARED((N, 16), f32))
    scratch += [
        pltpu.VMEM_SHARED((N, D), f32),
        pltpu.VMEM_SHARED((N, 16), f32),
        [pltpu.VMEM((CK,), i32)] * 3,
        [pltpu.VMEM((CK,), i32)] * 3,
        [pltpu.VMEM((CK, 16), f32)] * 2,
        [pltpu.VMEM((CK, 16), f32)] * 2,
        [pltpu.VMEM((CK, 16), f32)] * 2,
        [pltpu.VMEM((CK, D), f32)] * 2,
        [pltpu.SemaphoreType.DMA] * 16,
    ]
    kernel_deco = functools.partial(
        pl.kernel,
        out_type=[
            jax.ShapeDtypeStruct((2, N, D), f32),
            jax.ShapeDtypeStruct((2, N, 16), f32),
        ],
        mesh=mesh,
        compiler_params=pltpu.CompilerParams(use_tc_tiling_on_sc=False),
        scratch_types=scratch,
    )

    def impl(h_hbm, t1_hbm, t2_hbm, ninit_hbm, dinit_hbm,
             src_hbm, dst_hbm,
             nump_hbm, denp_hbm,
             sp_h, sp_t1, sp_t2, sp_num, sp_den,
             v_src, v_dst, v_s, v_d, v_ex, v_h, sems):
        cid = lax.axis_index("c")
        sid = lax.axis_index("s")
        wid = cid * 16 + sid
        rs = sid * RSTAGE
        iota = lax.iota(i32, 16)
        col_shift = jnp.where(iota < 8, iota + 8, iota)
        sp_td = sp_t2 if dual_t else sp_t1
        tile_base = wid * EPT
        (sem_is, sem_id, sem_g1, sem_g2, sem_gh, sem_sx, sem_sh) = (
            sems[0:3], sems[3:6], sems[6:8], sems[8:10], sems[10:12],
            sems[12:14], sems[14:16])

        def base(j):
            return pl.multiple_of(tile_base + j * CK, 8)

        def idx_copies(j, q):
            return (pltpu.make_async_copy(src_hbm.at[pl.ds(base(j), CK)],
                                          v_src[q], sem_is[q]),
                    pltpu.make_async_copy(dst_hbm.at[pl.ds(base(j), CK)],
                                          v_dst[q], sem_id[q]))

        def gather_copies(q, b):
            return (pltpu.make_async_copy(sp_t1.at[v_src[q]], v_s[b], sem_g1[b]),
                    pltpu.make_async_copy(sp_td.at[v_dst[q]], v_d[b], sem_g2[b]),
                    pltpu.make_async_copy(sp_h.at[v_src[q]], v_h[b], sem_gh[b]))

        def scatter_copies(q, b):
            return (pltpu.make_async_copy(v_ex[b], sp_den.at[v_dst[q]], sem_sx[b]),
                    pltpu.make_async_copy(v_h[b], sp_num.at[v_dst[q]], sem_sh[b]))

        def issue(copies, add=False):
            for c in copies:
                c.start(add=add)

        def wait(copies):
            for c in copies:
                c.wait()

        def compute(b):
            def rowbody(i, carry):
                d_row = v_d[b][i, :]
                if not dual_t:
                    d_row = d_row.at[col_shift].get(mode="promise_in_bounds")
                ex_row = jnp.exp(_leaky(v_s[b][i, :] + d_row))
                v_ex[b][i, :] = ex_row
                for j in range(NV):
                    if headed:
                        col = 2 * j + jnp.where(iota >= 8, 1, 0)
                        exb = ex_row.at[col].get(mode="promise_in_bounds")
                    else:
                        exb = ex_row
                    v_h[b][i, pl.ds(16 * j, 16)] = v_h[b][i, pl.ds(16 * j, 16)] * exb
                return carry

            lax.fori_loop(0, CK, rowbody, 0, unroll=16)

        def step(j, r, peeled_first=False):
            b = r % 2
            q = r % 3
            q1 = (r + 1) % 3
            q2 = (r + 2) % 3
            wait(gather_copies(q, b))
            if not peeled_first:
                wait(scatter_copies(q2, 1 - b))
            if isinstance(j, int):
                if j + 1 < NCHUNK:
                    wait(idx_copies(j + 1, q1))
                    issue(gather_copies(q1, 1 - b))
                if not peeled_first and j + 2 < NCHUNK:
                    issue(idx_copies(j + 2, q2))
            else:
                @pl.when(j + 1 < NCHUNK)
                def _():
                    wait(idx_copies(j + 1, q1))
                    issue(gather_copies(q1, 1 - b))

                @pl.when(j + 2 < NCHUNK)
                def _():
                    issue(idx_copies(j + 2, q2))

            compute(b)
            issue(scatter_copies(q, b), add=True)

        def serial_chunk(bs):
            bs = pl.multiple_of(bs, 8)
            pltpu.sync_copy(src_hbm.at[pl.ds(bs, CK)], v_src[0])
            pltpu.sync_copy(dst_hbm.at[pl.ds(bs, CK)], v_dst[0])
            wait_g = gather_copies(0, 0)
            issue(wait_g)
            wait(wait_g)
            compute(0)
            sc = scatter_copies(0, 0)
            issue(sc, add=True)
            wait(sc)

        @pl.when(sid < N // RSTAGE)
        def _():
            pltpu.sync_copy(h_hbm.at[pl.ds(rs, RSTAGE)], sp_h.at[pl.ds(rs, RSTAGE)])
            pltpu.sync_copy(t1_hbm.at[pl.ds(rs, RSTAGE)], sp_t1.at[pl.ds(rs, RSTAGE)])
            if dual_t:
                pltpu.sync_copy(t2_hbm.at[pl.ds(rs, RSTAGE)], sp_t2.at[pl.ds(rs, RSTAGE)])
            pltpu.sync_copy(ninit_hbm.at[pl.ds(rs, RSTAGE)], sp_num.at[pl.ds(rs, RSTAGE)])
            pltpu.sync_copy(dinit_hbm.at[pl.ds(rs, RSTAGE)], sp_den.at[pl.ds(rs, RSTAGE)])
        plsc.subcore_barrier()

        issue(idx_copies(0, 0))
        issue(idx_copies(1, 1))
        issue(idx_copies(2, 2))
        wait(idx_copies(0, 0))
        issue(gather_copies(0, 0))
        step(0, 0, peeled_first=True)
        for r in range(1, 6):
            step(r, r)

        def chunkloop(ii, carry):
            for r in range(6):
                step(6 * ii + r, r)
            return carry

        lax.fori_loop(1, NCHUNK // 6, chunkloop, 0)
        wait(scatter_copies((NCHUNK - 1) % 3, (NCHUNK - 1) % 2))

        @pl.when(wid < (E - NTILE * EPT) // CK)
        def _():
            serial_chunk(NTILE * EPT + wid * CK)

        plsc.subcore_barrier()

        @pl.when(sid < N // RSTAGE)
        def _():
            pltpu.sync_copy(sp_num.at[pl.ds(rs, RSTAGE)],
                            nump_hbm.at[cid, pl.ds(rs, RSTAGE)])
            pltpu.sync_copy(sp_den.at[pl.ds(rs, RSTAGE)],
                            denp_hbm.at[cid, pl.ds(rs, RSTAGE)])

    if dual_t:
        @kernel_deco
        def edge_kernel(h_hbm, t1_hbm, t2_hbm, ninit_hbm, dinit_hbm,
                        src_hbm, dst_hbm, nump_hbm, denp_hbm,
                        sp_h, sp_t1, sp_t2, sp_num, sp_den,
                        v_src, v_dst, v_s, v_d, v_ex, v_h, sems):
            impl(h_hbm, t1_hbm, t2_hbm, ninit_hbm, dinit_hbm,
                 src_hbm, dst_hbm, nump_hbm, denp_hbm,
                 sp_h, sp_t1, sp_t2, sp_num, sp_den,
                 v_src, v_dst, v_s, v_d, v_ex, v_h, sems)
    else:
        @kernel_deco
        def edge_kernel(h_hbm, t1_hbm, ninit_hbm, dinit_hbm,
                        src_hbm, dst_hbm, nump_hbm, denp_hbm,
                        sp_h, sp_t1, sp_num, sp_den,
                        v_src, v_dst, v_s, v_d, v_ex, v_h, sems):
            impl(h_hbm, t1_hbm, None, ninit_hbm, dinit_hbm,
                 src_hbm, dst_hbm, nump_hbm, denp_hbm,
                 sp_h, sp_t1, None, sp_num, sp_den,
                 v_src, v_dst, v_s, v_d, v_ex, v_h, sems)

    return edge_kernel


def _mid_body(np0_ref, dp0_ref, np1_ref, dp1_ref, ni_ref, di_ref, b1_ref,
              w2_ref, b2s_ref, b2d_ref, r8_ref,
              h2_ref, t1_ref, t2_ref, ninit2_ref, dinit2_ref):
    outs = []
    for k, (np_ref, dp_ref) in enumerate(((np0_ref, dp0_ref),
                                          (np1_ref, dp1_ref))):
        num = np_ref[0] + np_ref[1] - ni_ref[k]
        den8 = (dp_ref[0] + dp_ref[1] - di_ref[k])[:, :8]
        den = jnp.dot(den8, r8_ref[...], preferred_element_type=f32)
        o = num / (den + 1e-16) + b1_ref[k, :][None, :]
        outs.append(_elu(o))
    x2 = jnp.concatenate(outs, axis=1)
    h2 = jnp.dot(x2, w2_ref[...], preferred_element_type=f32)
    t1 = jnp.dot(h2, b2s_ref[...], preferred_element_type=f32)
    t2 = jnp.dot(h2, b2d_ref[...], preferred_element_type=f32)
    exl = jnp.exp(_leaky(t1[:, 0:1] + t2[:, 0:1]))
    h2_ref[...] = h2
    t1_ref[...] = t1
    t2_ref[...] = t2
    ninit2_ref[...] = h2 * exl
    dinit2_ref[...] = jnp.broadcast_to(exl, exl.shape[:1] + (16,))


def _mid(nump0, denp0, nump1, denp1, ninit, dinit, b1, W2p, B2S, B2D, R8):
    return pl.pallas_call(
        _mid_body,
        grid=(NRB,),
        in_specs=[
            pl.BlockSpec((2, RB, HC), lambda i: (0, i, 0)),
            pl.BlockSpec((2, RB, 16), lambda i: (0, i, 0)),
            pl.BlockSpec((2, RB, HC), lambda i: (0, i, 0)),
            pl.BlockSpec((2, RB, 16), lambda i: (0, i, 0)),
            pl.BlockSpec((2, RB, HC), lambda i: (0, i, 0)),
            pl.BlockSpec((2, RB, 16), lambda i: (0, i, 0)),
            pl.BlockSpec((2, HC), lambda i: (0, 0)),
            pl.BlockSpec((F_IN, D2), lambda i: (0, 0)),
            pl.BlockSpec((D2, 16), lambda i: (0, 0)),
            pl.BlockSpec((D2, 16), lambda i: (0, 0)),
            pl.BlockSpec((H, HC), lambda i: (0, 0)),
        ],
        out_specs=[
            pl.BlockSpec((RB, D2), lambda i: (i, 0)),
            pl.BlockSpec((RB, 16), lambda i: (i, 0)),
            pl.BlockSpec((RB, 16), lambda i: (i, 0)),
            pl.BlockSpec((RB, D2), lambda i: (i, 0)),
            pl.BlockSpec((RB, 16), lambda i: (i, 0)),
        ],
        out_shape=[
            jax.ShapeDtypeStruct((N, D2), f32),
            jax.ShapeDtypeStruct((N, 16), f32),
            jax.ShapeDtypeStruct((N, 16), f32),
            jax.ShapeDtypeStruct((N, D2), f32),
            jax.ShapeDtypeStruct((N, 16), f32),
        ],
    )(nump0, denp0, nump1, denp1, ninit, dinit, b1, W2p, B2S, B2D, R8)


def _final_body(np_ref, dp_ref, ni_ref, di_ref, b2_ref, out_ref):
    num = (np_ref[0] + np_ref[1] - ni_ref[...])[:, :NCLASS]
    den = (dp_ref[0] + dp_ref[1] - di_ref[...])[:, :1]
    out_ref[...] = num / (den + 1e-16) + b2_ref[0, :][None, :]


def _final(nump2, denp2, ninit2, dinit2, b2):
    return pl.pallas_call(
        _final_body,
        grid=(NRB,),
        in_specs=[
            pl.BlockSpec((2, RB, D2), lambda i: (0, i, 0)),
            pl.BlockSpec((2, RB, 16), lambda i: (0, i, 0)),
            pl.BlockSpec((RB, D2), lambda i: (i, 0)),
            pl.BlockSpec((RB, 16), lambda i: (i, 0)),
            pl.BlockSpec((1, NCLASS), lambda i: (0, 0)),
        ],
        out_specs=pl.BlockSpec((RB, NCLASS), lambda i: (i, 0)),
        out_shape=jax.ShapeDtypeStruct((N, NCLASS), f32),
    )(nump2, denp2, ninit2, dinit2, b2)


_edges1 = _make_edge_kernel(HC, True, False)
_edges2 = _make_edge_kernel(D2, False, True)


def kernel(x_list, edge_index, W1, att_src1, att_dst1, b1, W2, att_src2, att_dst2, b2):
    src = edge_index[0]
    dst = edge_index[1]

    eye8 = jnp.eye(8, dtype=f32)
    A1s = (att_src1[:, :, :, None] * eye8[None, :, None, :]).reshape(2, HC, H)
    A1d = (att_dst1[:, :, :, None] * eye8[None, :, None, :]).reshape(2, HC, H)
    BT1 = jnp.concatenate([A1s, A1d], axis=2)
    R8 = jnp.broadcast_to(eye8[:, :, None], (H, H, C)).reshape(H, HC)
    W2p = jnp.pad(W2, ((0, 0), (0, D2 - NCLASS)))
    a2s = jnp.pad(att_src2.reshape(-1), (0, D2 - NCLASS))
    a2d = jnp.pad(att_dst2.reshape(-1), (0, D2 - NCLASS))
    ones16 = jnp.ones((16,), f32)
    B2S = a2s[:, None] * ones16[None, :]
    B2D = a2d[:, None] * ones16[None, :]

    h1, t1, ninit, dinit = _prep1(x_list, W1, BT1, R8)
    nump0, denp0 = _edges1(h1[0], t1[0], ninit[0], dinit[0], src, dst)
    nump1, denp1 = _edges1(h1[1], t1[1], ninit[1], dinit[1], src, dst)
    h2, t1b, t2b, ninit2, dinit2 = _mid(nump0, denp0, nump1, denp1,
                                        ninit, dinit, b1, W2p, B2S, B2D, R8)
    nump2, denp2 = _edges2(h2, t1b, t2b, ninit2, dinit2, src, dst)
    return _final(nump2, denp2, ninit2, dinit2, b2.reshape(1, NCLASS))


# speedup vs baseline: 1.2079x; 1.2079x over previous
import functools

import jax
import jax.numpy as jnp
from jax import lax
from jax.experimental import pallas as pl
from jax.experimental.pallas import tpu as pltpu
from jax.experimental.pallas import tpu_sc as plsc

N = 10000
E = 320000
F_IN = 128
H = 8
C = 8
HC = 64
D2 = 48
NCLASS = 40
RB = 1000
NRB = N // RB
NTILE = 32
EPT = 9984
CK = 128
NCHUNK = EPT // CK
RSTAGE = 1000

f32 = jnp.float32
i32 = jnp.int32


def _leaky(x):
    return jnp.maximum(x, 0.2 * x)


def _elu(x):
    return jnp.where(x > 0, x, 0.2 * (jnp.exp(x) - 1.0))


def _prep1_body(x_ref, w_ref, bt1_ref, r8_ref,
                h_ref, t1_ref, ninit_ref, dinit_ref):
    h = jnp.dot(x_ref[0], w_ref[0], preferred_element_type=f32)
    t1 = jnp.dot(h, bt1_ref[0], preferred_element_type=f32)
    exl = jnp.exp(_leaky(t1[:, :8] + t1[:, 8:]))
    h_ref[0] = h
    t1_ref[0] = t1
    ninit_ref[0] = h * jnp.dot(exl, r8_ref[...], preferred_element_type=f32)
    dinit_ref[0] = jnp.concatenate([exl, exl], axis=1)


def _prep1(x_list, W1, BT1, R8):
    return pl.pallas_call(
        _prep1_body,
        grid=(2, NRB),
        in_specs=[
            pl.BlockSpec((1, RB, F_IN), lambda k, i: (k, i, 0)),
            pl.BlockSpec((1, F_IN, HC), lambda k, i: (k, 0, 0)),
            pl.BlockSpec((1, HC, 16), lambda k, i: (k, 0, 0)),
            pl.BlockSpec((H, HC), lambda k, i: (0, 0)),
        ],
        out_specs=[
            pl.BlockSpec((1, RB, HC), lambda k, i: (k, i, 0)),
            pl.BlockSpec((1, RB, 16), lambda k, i: (k, i, 0)),
            pl.BlockSpec((1, RB, HC), lambda k, i: (k, i, 0)),
            pl.BlockSpec((1, RB, 16), lambda k, i: (k, i, 0)),
        ],
        out_shape=[
            jax.ShapeDtypeStruct((2, N, HC), f32),
            jax.ShapeDtypeStruct((2, N, 16), f32),
            jax.ShapeDtypeStruct((2, N, HC), f32),
            jax.ShapeDtypeStruct((2, N, 16), f32),
        ],
    )(x_list, W1, BT1, R8)


def _make_edge_kernel(D, headed, dual_t):
    NV = D // 16
    mesh = plsc.VectorSubcoreMesh(core_axis_name="c", subcore_axis_name="s")

    scratch = [
        pltpu.VMEM_SHARED((N, D), f32),
        pltpu.VMEM_SHARED((N, 16), f32),
    ]
    if dual_t:
        scratch.append(pltpu.VMEM_SHARED((N, 16), f32))
    scratch += [
        pltpu.VMEM_SHARED((N, D), f32),
        pltpu.VMEM_SHARED((N, 16), f32),
        [pltpu.VMEM((CK,), i32)] * 3,
        [pltpu.VMEM((CK,), i32)] * 3,
        [pltpu.VMEM((CK, 16), f32)] * 2,
        [pltpu.VMEM((CK, 16), f32)] * 2,
        [pltpu.VMEM((CK, 16), f32)] * 2,
        [pltpu.VMEM((CK, D), f32)] * 2,
        [pltpu.SemaphoreType.DMA] * 16,
    ]
    kernel_deco = functools.partial(
        pl.kernel,
        out_type=[
            jax.ShapeDtypeStruct((2, N, D), f32),
            jax.ShapeDtypeStruct((2, N, 16), f32),
        ],
        mesh=mesh,
        compiler_params=pltpu.CompilerParams(use_tc_tiling_on_sc=False),
        scratch_types=scratch,
    )

    def impl(h_hbm, t1_hbm, t2_hbm, ninit_hbm, dinit_hbm,
             src_hbm, dst_hbm,
             nump_hbm, denp_hbm,
             sp_h, sp_t1, sp_t2, sp_num, sp_den,
             v_src, v_dst, v_s, v_d, v_ex, v_h, sems):
        cid = lax.axis_index("c")
        sid = lax.axis_index("s")
        wid = cid * 16 + sid
        rs = sid * RSTAGE
        iota = lax.iota(i32, 16)
        col_shift = jnp.where(iota < 8, iota + 8, iota)
        sp_td = sp_t2 if dual_t else sp_t1
        tile_base = wid * EPT
        (sem_is, sem_id, sem_g1, sem_g2, sem_gh, sem_sx, sem_sh) = (
            sems[0:3], sems[3:6], sems[6:8], sems[8:10], sems[10:12],
            sems[12:14], sems[14:16])

        def base(j):
            return pl.multiple_of(tile_base + j * CK, 8)

        def idx_copies(j, q):
            return (pltpu.make_async_copy(src_hbm.at[pl.ds(base(j), CK)],
                                          v_src[q], sem_is[q]),
                    pltpu.make_async_copy(dst_hbm.at[pl.ds(base(j), CK)],
                                          v_dst[q], sem_id[q]))

        def gather_copies(q, b):
            return (pltpu.make_async_copy(sp_t1.at[v_src[q]], v_s[b], sem_g1[b]),
                    pltpu.make_async_copy(sp_td.at[v_dst[q]], v_d[b], sem_g2[b]),
                    pltpu.make_async_copy(sp_h.at[v_src[q]], v_h[b], sem_gh[b]))

        def scatter_copies(q, b):
            return (pltpu.make_async_copy(v_ex[b], sp_den.at[v_dst[q]], sem_sx[b]),
                    pltpu.make_async_copy(v_h[b], sp_num.at[v_dst[q]], sem_sh[b]))

        def issue(copies, add=False):
            for c in copies:
                c.start(add=add)

        def wait(copies):
            for c in copies:
                c.wait()

        def compute(b):
            def rowbody(i, carry):
                d_row = v_d[b][i, :]
                if not dual_t:
                    d_row = d_row.at[col_shift].get(mode="promise_in_bounds")
                ex_row = jnp.exp(_leaky(v_s[b][i, :] + d_row))
                v_ex[b][i, :] = ex_row
                for j in range(NV):
                    if headed:
                        col = 2 * j + jnp.where(iota >= 8, 1, 0)
                        exb = ex_row.at[col].get(mode="promise_in_bounds")
                    else:
                        exb = ex_row
                    v_h[b][i, pl.ds(16 * j, 16)] = v_h[b][i, pl.ds(16 * j, 16)] * exb
                return carry

            lax.fori_loop(0, CK, rowbody, 0, unroll=8)

        def step(j, r, peeled_first=False):
            b = r % 2
            q = r % 3
            q1 = (r + 1) % 3
            q2 = (r + 2) % 3
            wait(gather_copies(q, b))
            if not peeled_first:
                wait(scatter_copies(q2, 1 - b))
            if isinstance(j, int):
                if j + 1 < NCHUNK:
                    wait(idx_copies(j + 1, q1))
                    issue(gather_copies(q1, 1 - b))
                if not peeled_first and j + 2 < NCHUNK:
                    issue(idx_copies(j + 2, q2))
            else:
                @pl.when(j + 1 < NCHUNK)
                def _():
                    wait(idx_copies(j + 1, q1))
                    issue(gather_copies(q1, 1 - b))

                @pl.when(j + 2 < NCHUNK)
                def _():
                    issue(idx_copies(j + 2, q2))

            compute(b)
            issue(scatter_copies(q, b), add=True)

        def serial_chunk(bs):
            bs = pl.multiple_of(bs, 8)
            pltpu.sync_copy(src_hbm.at[pl.ds(bs, CK)], v_src[0])
            pltpu.sync_copy(dst_hbm.at[pl.ds(bs, CK)], v_dst[0])
            wait_g = gather_copies(0, 0)
            issue(wait_g)
            wait(wait_g)
            compute(0)
            sc = scatter_copies(0, 0)
            issue(sc, add=True)
            wait(sc)

        @pl.when(sid < N // RSTAGE)
        def _():
            pltpu.sync_copy(h_hbm.at[pl.ds(rs, RSTAGE)], sp_h.at[pl.ds(rs, RSTAGE)])
            pltpu.sync_copy(t1_hbm.at[pl.ds(rs, RSTAGE)], sp_t1.at[pl.ds(rs, RSTAGE)])
            if dual_t:
                pltpu.sync_copy(t2_hbm.at[pl.ds(rs, RSTAGE)], sp_t2.at[pl.ds(rs, RSTAGE)])
            pltpu.sync_copy(ninit_hbm.at[pl.ds(rs, RSTAGE)], sp_num.at[pl.ds(rs, RSTAGE)])
            pltpu.sync_copy(dinit_hbm.at[pl.ds(rs, RSTAGE)], sp_den.at[pl.ds(rs, RSTAGE)])
        plsc.subcore_barrier()

        issue(idx_copies(0, 0))
        issue(idx_copies(1, 1))
        issue(idx_copies(2, 2))
        wait(idx_copies(0, 0))
        issue(gather_copies(0, 0))
        step(0, 0, peeled_first=True)
        for r in range(1, 6):
            step(r, r)

        def chunkloop(ii, carry):
            for r in range(6):
                step(6 * ii + r, r)
            return carry

        lax.fori_loop(1, NCHUNK // 6, chunkloop, 0)
        wait(scatter_copies((NCHUNK - 1) % 3, (NCHUNK - 1) % 2))

        @pl.when(wid < (E - NTILE * EPT) // CK)
        def _():
            serial_chunk(NTILE * EPT + wid * CK)

        plsc.subcore_barrier()

        @pl.when(sid < N // RSTAGE)
        def _():
            pltpu.sync_copy(sp_num.at[pl.ds(rs, RSTAGE)],
                            nump_hbm.at[cid, pl.ds(rs, RSTAGE)])
            pltpu.sync_copy(sp_den.at[pl.ds(rs, RSTAGE)],
                            denp_hbm.at[cid, pl.ds(rs, RSTAGE)])

    if dual_t:
        @kernel_deco
        def edge_kernel(h_hbm, t1_hbm, t2_hbm, ninit_hbm, dinit_hbm,
                        src_hbm, dst_hbm, nump_hbm, denp_hbm,
                        sp_h, sp_t1, sp_t2, sp_num, sp_den,
                        v_src, v_dst, v_s, v_d, v_ex, v_h, sems):
            impl(h_hbm, t1_hbm, t2_hbm, ninit_hbm, dinit_hbm,
                 src_hbm, dst_hbm, nump_hbm, denp_hbm,
                 sp_h, sp_t1, sp_t2, sp_num, sp_den,
                 v_src, v_dst, v_s, v_d, v_ex, v_h, sems)
    else:
        @kernel_deco
        def edge_kernel(h_hbm, t1_hbm, ninit_hbm, dinit_hbm,
                        src_hbm, dst_hbm, nump_hbm, denp_hbm,
                        sp_h, sp_t1, sp_num, sp_den,
                        v_src, v_dst, v_s, v_d, v_ex, v_h, sems):
            impl(h_hbm, t1_hbm, None, ninit_hbm, dinit_hbm,
                 src_hbm, dst_hbm, nump_hbm, denp_hbm,
                 sp_h, sp_t1, None, sp_num, sp_den,
                 v_src, v_dst, v_s, v_d, v_ex, v_h, sems)

    return edge_kernel


def _mid_body(np0_ref, dp0_ref, np1_ref, dp1_ref, ni_ref, di_ref, b1_ref,
              w2_ref, b2s_ref, b2d_ref, r8_ref,
              h2_ref, t1_ref, t2_ref, ninit2_ref, dinit2_ref):
    outs = []
    for k, (np_ref, dp_ref) in enumerate(((np0_ref, dp0_ref),
                                          (np1_ref, dp1_ref))):
        num = np_ref[0] + np_ref[1] - ni_ref[k]
        den8 = (dp_ref[0] + dp_ref[1] - di_ref[k])[:, :8]
        den = jnp.dot(den8, r8_ref[...], preferred_element_type=f32)
        o = num / (den + 1e-16) + b1_ref[k, :][None, :]
        outs.append(_elu(o))
    x2 = jnp.concatenate(outs, axis=1)
    h2 = jnp.dot(x2, w2_ref[...], preferred_element_type=f32)
    t1 = jnp.dot(h2, b2s_ref[...], preferred_element_type=f32)
    t2 = jnp.dot(h2, b2d_ref[...], preferred_element_type=f32)
    exl = jnp.exp(_leaky(t1[:, 0:1] + t2[:, 0:1]))
    h2_ref[...] = h2
    t1_ref[...] = t1
    t2_ref[...] = t2
    ninit2_ref[...] = h2 * exl
    dinit2_ref[...] = jnp.broadcast_to(exl, exl.shape[:1] + (16,))


def _mid(nump0, denp0, nump1, denp1, ninit, dinit, b1, W2p, B2S, B2D, R8):
    return pl.pallas_call(
        _mid_body,
        grid=(NRB,),
        in_specs=[
            pl.BlockSpec((2, RB, HC), lambda i: (0, i, 0)),
            pl.BlockSpec((2, RB, 16), lambda i: (0, i, 0)),
            pl.BlockSpec((2, RB, HC), lambda i: (0, i, 0)),
            pl.BlockSpec((2, RB, 16), lambda i: (0, i, 0)),
            pl.BlockSpec((2, RB, HC), lambda i: (0, i, 0)),
            pl.BlockSpec((2, RB, 16), lambda i: (0, i, 0)),
            pl.BlockSpec((2, HC), lambda i: (0, 0)),
            pl.BlockSpec((F_IN, D2), lambda i: (0, 0)),
            pl.BlockSpec((D2, 16), lambda i: (0, 0)),
            pl.BlockSpec((D2, 16), lambda i: (0, 0)),
            pl.BlockSpec((H, HC), lambda i: (0, 0)),
        ],
        out_specs=[
            pl.BlockSpec((RB, D2), lambda i: (i, 0)),
            pl.BlockSpec((RB, 16), lambda i: (i, 0)),
            pl.BlockSpec((RB, 16), lambda i: (i, 0)),
            pl.BlockSpec((RB, D2), lambda i: (i, 0)),
            pl.BlockSpec((RB, 16), lambda i: (i, 0)),
        ],
        out_shape=[
            jax.ShapeDtypeStruct((N, D2), f32),
            jax.ShapeDtypeStruct((N, 16), f32),
            jax.ShapeDtypeStruct((N, 16), f32),
            jax.ShapeDtypeStruct((N, D2), f32),
            jax.ShapeDtypeStruct((N, 16), f32),
        ],
    )(nump0, denp0, nump1, denp1, ninit, dinit, b1, W2p, B2S, B2D, R8)


def _final_body(np_ref, dp_ref, ni_ref, di_ref, b2_ref, out_ref):
    num = (np_ref[0] + np_ref[1] - ni_ref[...])[:, :NCLASS]
    den = (dp_ref[0] + dp_ref[1] - di_ref[...])[:, :1]
    out_ref[...] = num / (den + 1e-16) + b2_ref[0, :][None, :]


def _final(nump2, denp2, ninit2, dinit2, b2):
    return pl.pallas_call(
        _final_body,
        grid=(NRB,),
        in_specs=[
            pl.BlockSpec((2, RB, D2), lambda i: (0, i, 0)),
            pl.BlockSpec((2, RB, 16), lambda i: (0, i, 0)),
            pl.BlockSpec((RB, D2), lambda i: (i, 0)),
            pl.BlockSpec((RB, 16), lambda i: (i, 0)),
            pl.BlockSpec((1, NCLASS), lambda i: (0, 0)),
        ],
        out_specs=pl.BlockSpec((RB, NCLASS), lambda i: (i, 0)),
        out_shape=jax.ShapeDtypeStruct((N, NCLASS), f32),
    )(nump2, denp2, ninit2, dinit2, b2)


_edges1 = _make_edge_kernel(HC, True, False)
_edges2 = _make_edge_kernel(D2, False, True)


def kernel(x_list, edge_index, W1, att_src1, att_dst1, b1, W2, att_src2, att_dst2, b2):
    src = edge_index[0]
    dst = edge_index[1]

    eye8 = jnp.eye(8, dtype=f32)
    A1s = (att_src1[:, :, :, None] * eye8[None, :, None, :]).reshape(2, HC, H)
    A1d = (att_dst1[:, :, :, None] * eye8[None, :, None, :]).reshape(2, HC, H)
    BT1 = jnp.concatenate([A1s, A1d], axis=2)
    R8 = jnp.broadcast_to(eye8[:, :, None], (H, H, C)).reshape(H, HC)
    W2p = jnp.pad(W2, ((0, 0), (0, D2 - NCLASS)))
    a2s = jnp.pad(att_src2.reshape(-1), (0, D2 - NCLASS))
    a2d = jnp.pad(att_dst2.reshape(-1), (0, D2 - NCLASS))
    ones16 = jnp.ones((16,), f32)
    B2S = a2s[:, None] * ones16[None, :]
    B2D = a2d[:, None] * ones16[None, :]

    h1, t1, ninit, dinit = _prep1(x_list, W1, BT1, R8)
    nump0, denp0 = _edges1(h1[0], t1[0], ninit[0], dinit[0], src, dst)
    nump1, denp1 = _edges1(h1[1], t1[1], ninit[1], dinit[1], src, dst)
    h2, t1b, t2b, ninit2, dinit2 = _mid(nump0, denp0, nump1, denp1,
                                        ninit, dinit, b1, W2p, B2S, B2D, R8)
    nump2, denp2 = _edges2(h2, t1b, t2b, ninit2, dinit2, src, dst)
    return _final(nump2, denp2, ninit2, dinit2, b2.reshape(1, NCLASS))
